# baseline (speedup 1.0000x reference)
import functools

import jax
import jax.numpy as jnp
from jax import lax
from jax.experimental import pallas as pl
from jax.experimental.pallas import tpu as pltpu
from jax.experimental.pallas import tpu_sc as plsc

N_GRAPHS = 16
NP = 10240
NROWS = NP // 128
F = 128
E = 320000
NC, NS = 2, 16
EDGES_PER_TILE = E // (NC * NS)
CHUNK = 128
NFULL = EDGES_PER_TILE // CHUNK
REM = EDGES_PER_TILE - NFULL * CHUNK
ROWS_PER_TILE = NP // NS
STAGE = 24
INT_MIN32 = -2147483648


def _sc_segment_sum(y, src2d, dst2d, srct, dstt):
    mesh = plsc.VectorSubcoreMesh(
        core_axis_name="c", subcore_axis_name="s", num_cores=NC,
        num_subcores=NS)

    @functools.partial(
        pl.kernel,
        out_type=jax.ShapeDtypeStruct((NC, NP, F), jnp.float32),
        mesh=mesh,
        scratch_types=[
            pltpu.VMEM((STAGE, CHUNK), jnp.int32),
            pltpu.VMEM((STAGE, CHUNK), jnp.int32),
            pltpu.VMEM((CHUNK, F), jnp.float32),
            pltpu.VMEM((CHUNK, F), jnp.float32),
            pltpu.VMEM((REM,), jnp.int32),
            pltpu.VMEM((REM,), jnp.int32),
            pltpu.VMEM((16, F), jnp.float32),
            pltpu.VMEM_SHARED((NP, F), jnp.float32),
            pltpu.SemaphoreType.DMA,
            pltpu.SemaphoreType.DMA,
            pltpu.SemaphoreType.DMA,
            pltpu.SemaphoreType.DMA,
            pltpu.SemaphoreType.DMA,
        ],
    )
    def k(y_hbm, src_hbm, dst_hbm, srct_hbm, dstt_hbm, out_hbm,
          sidx, didx, rows0, rows1, sidxr, didxr, zbuf, acc,
          sg0, sg1, ss0, ss1, semi):
        c = lax.axis_index("c")
        s = lax.axis_index("s")
        w = c * NS + s

        i0 = pltpu.make_async_copy(src_hbm.at[w, pl.ds(0, STAGE)], sidx, semi)
        i1 = pltpu.make_async_copy(dst_hbm.at[w, pl.ds(0, STAGE)], didx, semi)
        i2 = pltpu.make_async_copy(srct_hbm.at[pl.ds(w * REM, REM)],
                                   sidxr, semi)
        i3 = pltpu.make_async_copy(dstt_hbm.at[pl.ds(w * REM, REM)],
                                   didxr, semi)
        i0.start(); i1.start(); i2.start(); i3.start()

        for i in range(16):
            for j in range(F // 16):
                zbuf[i, pl.ds(j * 16, 16)] = jnp.zeros((16,), jnp.float32)
        row0 = s * ROWS_PER_TILE

        def zero_body(kk, _):
            pltpu.sync_copy(zbuf, acc.at[pl.ds(row0 + kk * 16, 16)])
            return _
        lax.fori_loop(0, ROWS_PER_TILE // 16, zero_body, 0)
        i0.wait(); i1.wait(); i2.wait(); i3.wait()
        plsc.subcore_barrier()

        def gather(kk, buf, sg):
            return pltpu.make_async_copy(y_hbm.at[sidx.at[kk]], buf, sg)

        def run_stage(npairs):
            gather(0, rows0, sg0).start()
            gather(1, rows1, sg1).start()

            def pair_body(kk, _):
                e = 2 * kk
                gather(e, rows0, sg0).wait()
                d0 = pltpu.async_copy(rows0, acc.at[didx.at[e]], ss0,
                                      add=True)
                gather(e + 1, rows1, sg1).wait()
                d1 = pltpu.async_copy(rows1, acc.at[didx.at[e + 1]], ss1,
                                      add=True)
                d0.wait()

                @pl.when(kk < npairs - 1)
                def _next0():
                    gather(e + 2, rows0, sg0).start()

                d1.wait()

                @pl.when(kk < npairs - 1)
                def _next1():
                    gather(e + 3, rows1, sg1).start()

                return _
            lax.fori_loop(0, npairs, pair_body, 0)

        for t in range(NFULL // STAGE):
            if t > 0:
                pltpu.sync_copy(src_hbm.at[w, pl.ds(t * STAGE, STAGE)], sidx)
                pltpu.sync_copy(dst_hbm.at[w, pl.ds(t * STAGE, STAGE)], didx)
            run_stage(STAGE // 2)
        tcount = NFULL - (NFULL // STAGE) * STAGE
        pltpu.sync_copy(
            src_hbm.at[w, pl.ds((NFULL // STAGE) * STAGE, tcount)],
            sidx.at[pl.ds(0, tcount)])
        pltpu.sync_copy(
            dst_hbm.at[w, pl.ds((NFULL // STAGE) * STAGE, tcount)],
            didx.at[pl.ds(0, tcount)])
        run_stage(tcount // 2)

        pltpu.async_copy(y_hbm.at[sidxr], rows1.at[pl.ds(0, REM)], sg1).wait()
        pltpu.sync_copy(rows1.at[pl.ds(0, REM)], acc.at[didxr], add=True)

        plsc.subcore_barrier()
        pltpu.sync_copy(acc.at[pl.ds(row0, ROWS_PER_TILE)],
                        out_hbm.at[c].at[pl.ds(row0, ROWS_PER_TILE)])

    return k(y, src2d, dst2d, srct, dstt)


_BLK = 512
_GRID = NP // _BLK


def _row_spec(blk=_BLK, width=F):
    return pl.BlockSpec((blk, width), lambda i: (i, 0))


def _full_spec(shape):
    return pl.BlockSpec(shape, lambda i: tuple(0 for _ in shape))


def _k0_body(x_ref, w_ref, o_ref):
    o_ref[...] = jnp.dot(x_ref[...], w_ref[...],
                         preferred_element_type=jnp.float32)


def _matmul(x, w):
    return pl.pallas_call(
        _k0_body,
        grid=(_GRID,),
        in_specs=[_row_spec(), _full_spec((F, F))],
        out_specs=_row_spec(),
        out_shape=jax.ShapeDtypeStruct((NP, F), jnp.float32),
    )(x, w)


def _k1_body(p0_ref, p1_ref, h_ref, wroot_ref, b_ref, wrel2_ref,
             h1_ref, y2_ref):
    agg = p0_ref[...] + p1_ref[...]
    h1 = jnp.maximum(
        agg + jnp.dot(h_ref[...], wroot_ref[...],
                      preferred_element_type=jnp.float32) + b_ref[...], 0.0)
    h1_ref[...] = h1
    y2_ref[...] = jnp.dot(h1, wrel2_ref[...],
                          preferred_element_type=jnp.float32)


def _combine_and_next(p0, p1, h, wroot, b, wrel2):
    return pl.pallas_call(
        _k1_body,
        grid=(_GRID,),
        in_specs=[_row_spec(), _row_spec(), _row_spec(),
                  _full_spec((F, F)), _full_spec((1, F)), _full_spec((F, F))],
        out_specs=[_row_spec(), _row_spec()],
        out_shape=[jax.ShapeDtypeStruct((NP, F), jnp.float32),
                   jax.ShapeDtypeStruct((NP, F), jnp.float32)],
    )(p0, p1, h, wroot, b, wrel2)


def _k2_body(p0_ref, p1_ref, h_ref, wroot_ref, b_ref, pcol_ref,
             h2_ref, q_ref):
    agg = p0_ref[...] + p1_ref[...]
    h2 = jnp.maximum(
        agg + jnp.dot(h_ref[...], wroot_ref[...],
                      preferred_element_type=jnp.float32) + b_ref[...], 0.0)
    h2_ref[...] = h2
    q_ref[...] = jnp.dot(h2, pcol_ref[...],
                         preferred_element_type=jnp.float32)


def _combine_and_score(p0, p1, h, wroot, b, pcol):
    return pl.pallas_call(
        _k2_body,
        grid=(_GRID,),
        in_specs=[_row_spec(), _row_spec(), _row_spec(),
                  _full_spec((F, F)), _full_spec((1, F)), _full_spec((F, 1))],
        out_specs=[_row_spec(), pl.BlockSpec((_BLK, 1), lambda i: (i, 0))],
        out_shape=[jax.ShapeDtypeStruct((NP, F), jnp.float32),
                   jax.ShapeDtypeStruct((NP, 1), jnp.float32)],
    )(p0, p1, h, wroot, b, pcol)


def _pool_body(q_ref, batch_ref, pp_ref, s_ref, keep_ref, counts_ref):
    q = q_ref[...]
    batch = batch_ref[...]
    bits = lax.bitcast_convert_type(q, jnp.int32)
    key = bits ^ (lax.shift_right_arithmetic(bits, 31) & jnp.int32(0x7FFFFFFF))

    mg = [batch == g for g in range(N_GRAPHS)]
    mgf = [m.astype(jnp.float32) for m in mg]
    ones_col = jnp.ones((128, 1), jnp.float32)

    def select_chain(scalars):
        t = jnp.broadcast_to(scalars[N_GRAPHS - 1], q.shape)
        for g in range(N_GRAPHS - 1):
            t = jnp.where(mg[g], scalars[g], t)
        return t

    def gcounts(predf, masksf):
        rows = [jnp.sum(predf * masksf[g], axis=0, keepdims=True)
                for g in range(N_GRAPHS)]
        return lax.dot_general(jnp.concatenate(rows, axis=0), ones_col,
                               (((1,), (0,)), ((), ())),
                               preferred_element_type=jnp.float32)

    onesf = jnp.ones(q.shape, jnp.float32)
    sizes = gcounts(onesf, mgf)
    kper = jnp.ceil(jnp.float32(0.8) * sizes)
    kper_i = kper.astype(jnp.int32)

    def t_body(i, tu):
        b = 31 - i
        bit = lax.shift_left(jnp.int32(1), b)
        cand_u = tu | bit
        cand_k = cand_u ^ jnp.int32(INT_MIN32)
        tn = select_chain([cand_k[g, 0] for g in range(N_GRAPHS)])
        predf = (key >= tn).astype(jnp.float32)
        cnt = gcounts(predf, mgf)
        return jnp.where(cnt >= kper, cand_u, tu)

    tu = lax.fori_loop(0, 32, t_body, jnp.zeros((N_GRAPHS, 1), jnp.int32))
    tk = tu ^ jnp.int32(INT_MIN32)

    tkn = select_chain([tk[g, 0] for g in range(N_GRAPHS)])
    tief = (key == tkn).astype(jnp.float32)
    wgf = [tief * mgf[g] for g in range(N_GRAPHS)]
    cgt = gcounts((key > tkn).astype(jnp.float32), mgf)
    m = kper - cgt

    idx = (lax.broadcasted_iota(jnp.int32, q.shape, 0) * 128
           + lax.broadcasted_iota(jnp.int32, q.shape, 1))

    def j_body(i, jv):
        b = 13 - i
        bit = lax.shift_left(jnp.int32(1), b)
        cand = jv | bit
        jn = select_chain([cand[g, 0] for g in range(N_GRAPHS)])
        cnt = gcounts((idx < jn).astype(jnp.float32), wgf)
        return jnp.where(cnt < m, cand, jv)

    jv = lax.fori_loop(0, 14, j_body, jnp.zeros((N_GRAPHS, 1), jnp.int32))

    jn = select_chain([jv[g, 0] for g in range(N_GRAPHS)])
    keep = (key > tkn) | ((key == tkn) & (idx <= jn))
    keep = keep & (batch < N_GRAPHS)
    keep_f = keep.astype(jnp.float32)

    pp = pp_ref[...]
    inv_norm = lax.rsqrt(jnp.sum(pp * pp))
    s_ref[...] = jnp.tanh(q * inv_norm) * keep_f
    keep_ref[...] = keep_f
    counts_ref[...] = gcounts(keep_f, mgf)


def _pool_select(q2, batch2, pp):
    return pl.pallas_call(
        _pool_body,
        grid=(1,),
        in_specs=[_full_spec((NROWS, 128)), _full_spec((NROWS, 128)),
                  _full_spec((1, F))],
        out_specs=[_full_spec((NROWS, 128)), _full_spec((NROWS, 128)),
                   _full_spec((N_GRAPHS, 1))],
        out_shape=[jax.ShapeDtypeStruct((NROWS, 128), jnp.float32),
                   jax.ShapeDtypeStruct((NROWS, 128), jnp.float32),
                   jax.ShapeDtypeStruct((N_GRAPHS, 1), jnp.float32)],
    )(q2, batch2, pp)


def _k3b_body(h2_ref, s_ref, wrel3_ref, h2p_ref, y3_ref):
    h2p = h2_ref[...] * s_ref[...]
    h2p_ref[...] = h2p
    y3_ref[...] = jnp.dot(h2p, wrel3_ref[...],
                          preferred_element_type=jnp.float32)


def _scale_and_next(h2, s, wrel3):
    return pl.pallas_call(
        _k3b_body,
        grid=(_GRID,),
        in_specs=[_row_spec(), pl.BlockSpec((_BLK, 1), lambda i: (i, 0)),
                  _full_spec((F, F))],
        out_specs=[_row_spec(), _row_spec()],
        out_shape=[jax.ShapeDtypeStruct((NP, F), jnp.float32),
                   jax.ShapeDtypeStruct((NP, F), jnp.float32)],
    )(h2, s, wrel3)


def _k45_body(p0_ref, p1_ref, h2p_ref, wroot_ref, b_ref, keep_ref,
              batch_ref, counts_ref, w1_ref, b1_ref, w2_ref, b2_ref,
              o_ref, gsum_ref):
    @pl.when(pl.program_id(0) == 0)
    def _init():
        gsum_ref[...] = jnp.zeros_like(gsum_ref)

    agg = p0_ref[...] + p1_ref[...]
    h3 = keep_ref[...] * jnp.maximum(
        agg + jnp.dot(h2p_ref[...], wroot_ref[...],
                      preferred_element_type=jnp.float32) + b_ref[...], 0.0)
    oh = (batch_ref[...] == lax.broadcasted_iota(
        jnp.int32, (_BLK, N_GRAPHS), 1)).astype(jnp.float32)
    gsum_ref[...] += lax.dot_general(
        oh, h3, (((0,), (0,)), ((), ())),
        preferred_element_type=jnp.float32)

    @pl.when(pl.program_id(0) == _GRID - 1)
    def _head():
        mean = gsum_ref[...] / jnp.maximum(counts_ref[...], 1.0)
        z = jnp.maximum(jnp.dot(mean, w1_ref[...],
                                preferred_element_type=jnp.float32)
                        + b1_ref[...], 0.0)
        logits = jnp.dot(z, w2_ref[...],
                         preferred_element_type=jnp.float32) + b2_ref[...]
        mx = jnp.max(logits, axis=1, keepdims=True)
        lse = jnp.log(jnp.sum(jnp.exp(logits - mx), axis=1, keepdims=True))
        o_ref[...] = logits - mx - lse


def _final_conv_pool_head(p0, p1, h2p, wroot, b, keep, batch_col,
                          counts, w1, b1, w2, b2):
    return pl.pallas_call(
        _k45_body,
        grid=(_GRID,),
        in_specs=[_row_spec(), _row_spec(), _row_spec(),
                  _full_spec((F, F)), _full_spec((1, F)),
                  pl.BlockSpec((_BLK, 1), lambda i: (i, 0)),
                  pl.BlockSpec((_BLK, 1), lambda i: (i, 0)),
                  _full_spec((N_GRAPHS, 1)),
                  _full_spec((F, 64)), _full_spec((1, 64)),
                  _full_spec((64, 10)), _full_spec((1, 10))],
        out_specs=_full_spec((N_GRAPHS, 10)),
        out_shape=jax.ShapeDtypeStruct((N_GRAPHS, 10), jnp.float32),
        scratch_shapes=[pltpu.VMEM((N_GRAPHS, F), jnp.float32)],
    )(p0, p1, h2p, wroot, b, keep, batch_col, counts, w1, b1, w2, b2)


def kernel(x, edge_index, batch,
           conv1_Wrel, conv1_brel, conv1_Wroot,
           conv2_Wrel, conv2_brel, conv2_Wroot,
           conv3_Wrel, conv3_brel, conv3_Wroot,
           pool_p, lin1_W, lin1_b, lin2_W, lin2_b):
    n = x.shape[0]
    xp = jnp.pad(x, ((0, NP - n), (0, 0)))
    src = edge_index[0].astype(jnp.int32)
    dst = edge_index[1].astype(jnp.int32)
    nfe = NFULL * CHUNK * NC * NS
    src2d = src[:nfe].reshape(NC * NS, NFULL, CHUNK)
    dst2d = dst[:nfe].reshape(NC * NS, NFULL, CHUNK)
    srct = src[nfe:]
    dstt = dst[nfe:]
    batch_pad = jnp.pad(batch.astype(jnp.int32), (0, NP - n),
                        constant_values=127)
    batch2 = batch_pad.reshape(NROWS, 128)
    batch_col = batch_pad.reshape(NP, 1)
    b1 = conv1_brel.reshape(1, F)
    b2 = conv2_brel.reshape(1, F)
    b3 = conv3_brel.reshape(1, F)
    pp_row = pool_p.reshape(1, F)
    pcol = pool_p.reshape(F, 1)

    y1 = _matmul(xp, conv1_Wrel)
    part1 = _sc_segment_sum(y1, src2d, dst2d, srct, dstt)
    h1, y2 = _combine_and_next(part1[0], part1[1], xp, conv1_Wroot, b1,
                               conv2_Wrel)
    part2 = _sc_segment_sum(y2, src2d, dst2d, srct, dstt)
    h2, q = _combine_and_score(part2[0], part2[1], h1, conv2_Wroot, b2, pcol)

    s2, keep2, counts = _pool_select(q.reshape(NROWS, 128), batch2, pp_row)
    s_col = s2.reshape(NP, 1)
    keep_col = keep2.reshape(NP, 1)

    h2p, y3 = _scale_and_next(h2, s_col, conv3_Wrel)
    part3 = _sc_segment_sum(y3, src2d, dst2d, srct, dstt)
    return _final_conv_pool_head(part3[0], part3[1], h2p, conv3_Wroot, b3,
                                 keep_col, batch_col, counts,
                                 lin1_W, lin1_b.reshape(1, 64),
                                 lin2_W, lin2_b.reshape(1, 10))


# speedup vs baseline: 1.1159x; 1.1159x over previous
import functools

import jax
import jax.numpy as jnp
from jax import lax
from jax.experimental import pallas as pl
from jax.experimental.pallas import tpu as pltpu
from jax.experimental.pallas import tpu_sc as plsc

N_GRAPHS = 16
NP = 10240
NROWS = NP // 128
F = 128
E = 320000
NC, NS = 2, 16
EDGES_PER_TILE = E // (NC * NS)
CHUNK = 128
NFULL = EDGES_PER_TILE // CHUNK
REM = EDGES_PER_TILE - NFULL * CHUNK
ROWS_PER_TILE = NP // NS
STAGE = 24
INT_MIN32 = -2147483648


def _sc_segment_sum(y, src2d, dst2d, srct, dstt):
    mesh = plsc.VectorSubcoreMesh(
        core_axis_name="c", subcore_axis_name="s", num_cores=NC,
        num_subcores=NS)

    @functools.partial(
        pl.kernel,
        out_type=jax.ShapeDtypeStruct((NC, NP, F), jnp.float32),
        mesh=mesh,
        scratch_types=[
            pltpu.VMEM((STAGE, CHUNK), jnp.int32),
            pltpu.VMEM((STAGE, CHUNK), jnp.int32),
            pltpu.VMEM((CHUNK, F), jnp.float32),
            pltpu.VMEM((CHUNK, F), jnp.float32),
            pltpu.VMEM((REM,), jnp.int32),
            pltpu.VMEM((REM,), jnp.int32),
            pltpu.VMEM((16, F), jnp.float32),
            pltpu.VMEM_SHARED((NP, F), jnp.float32),
            pltpu.SemaphoreType.DMA,
            pltpu.SemaphoreType.DMA,
            pltpu.SemaphoreType.DMA,
            pltpu.SemaphoreType.DMA,
            pltpu.SemaphoreType.DMA,
        ],
    )
    def k(y_hbm, src_hbm, dst_hbm, srct_hbm, dstt_hbm, out_hbm,
          sidx, didx, rows0, rows1, sidxr, didxr, zbuf, acc,
          sg0, sg1, ss0, ss1, semi):
        c = lax.axis_index("c")
        s = lax.axis_index("s")
        w = c * NS + s

        i0 = pltpu.make_async_copy(src_hbm.at[w, pl.ds(0, STAGE)], sidx, semi)
        i1 = pltpu.make_async_copy(dst_hbm.at[w, pl.ds(0, STAGE)], didx, semi)
        i2 = pltpu.make_async_copy(srct_hbm.at[pl.ds(w * REM, REM)],
                                   sidxr, semi)
        i3 = pltpu.make_async_copy(dstt_hbm.at[pl.ds(w * REM, REM)],
                                   didxr, semi)
        i0.start(); i1.start(); i2.start(); i3.start()

        for i in range(16):
            for j in range(F // 16):
                zbuf[i, pl.ds(j * 16, 16)] = jnp.zeros((16,), jnp.float32)
        row0 = s * ROWS_PER_TILE

        def zero_body(kk, _):
            pltpu.sync_copy(zbuf, acc.at[pl.ds(row0 + kk * 16, 16)])
            return _
        lax.fori_loop(0, ROWS_PER_TILE // 16, zero_body, 0)
        i0.wait(); i1.wait(); i2.wait(); i3.wait()
        plsc.subcore_barrier()

        def gather(kk, buf, sg):
            return pltpu.make_async_copy(y_hbm.at[sidx.at[kk]], buf, sg)

        def run_stage(npairs):
            gather(0, rows0, sg0).start()

            def pair_body(kk, _):
                e = 2 * kk
                gather(e, rows0, sg0).wait()
                gather(e + 1, rows1, sg1).start()
                pltpu.sync_copy(rows0, acc.at[didx.at[e]], add=True)
                gather(e + 1, rows1, sg1).wait()

                @pl.when(kk < npairs - 1)
                def _next():
                    gather(e + 2, rows0, sg0).start()

                pltpu.sync_copy(rows1, acc.at[didx.at[e + 1]], add=True)
                return _
            lax.fori_loop(0, npairs, pair_body, 0)

        for t in range(NFULL // STAGE):
            if t > 0:
                pltpu.sync_copy(src_hbm.at[w, pl.ds(t * STAGE, STAGE)], sidx)
                pltpu.sync_copy(dst_hbm.at[w, pl.ds(t * STAGE, STAGE)], didx)
            run_stage(STAGE // 2)
        tcount = NFULL - (NFULL // STAGE) * STAGE
        pltpu.sync_copy(
            src_hbm.at[w, pl.ds((NFULL // STAGE) * STAGE, tcount)],
            sidx.at[pl.ds(0, tcount)])
        pltpu.sync_copy(
            dst_hbm.at[w, pl.ds((NFULL // STAGE) * STAGE, tcount)],
            didx.at[pl.ds(0, tcount)])
        run_stage(tcount // 2)

        pltpu.async_copy(y_hbm.at[sidxr], rows1.at[pl.ds(0, REM)], sg1).wait()
        pltpu.sync_copy(rows1.at[pl.ds(0, REM)], acc.at[didxr], add=True)

        plsc.subcore_barrier()
        pltpu.sync_copy(acc.at[pl.ds(row0, ROWS_PER_TILE)],
                        out_hbm.at[c].at[pl.ds(row0, ROWS_PER_TILE)])

    return k(y, src2d, dst2d, srct, dstt)


_BLK = 512
_GRID = NP // _BLK


def _row_spec(blk=_BLK, width=F):
    return pl.BlockSpec((blk, width), lambda i: (i, 0))


def _part_spec(core):
    return pl.BlockSpec((1, _BLK, F), lambda i, c=core: (c, i, 0))


def _full_spec(shape):
    return pl.BlockSpec(shape, lambda i: tuple(0 for _ in shape))


def _k0_body(x_ref, w_ref, o_ref):
    o_ref[...] = jnp.dot(x_ref[...], w_ref[...],
                         preferred_element_type=jnp.float32)


def _matmul(x, w):
    return pl.pallas_call(
        _k0_body,
        grid=(_GRID,),
        in_specs=[_row_spec(), _full_spec((F, F))],
        out_specs=_row_spec(),
        out_shape=jax.ShapeDtypeStruct((NP, F), jnp.float32),
    )(x, w)


def _kr_body(h_ref, wroot_ref, b_ref, r_ref):
    r_ref[...] = jnp.dot(h_ref[...], wroot_ref[...],
                         preferred_element_type=jnp.float32) + b_ref[...]


def _root(h, wroot, b):
    return pl.pallas_call(
        _kr_body,
        grid=(_GRID,),
        in_specs=[_row_spec(), _full_spec((F, F)), _full_spec((1, F))],
        out_specs=_row_spec(),
        out_shape=jax.ShapeDtypeStruct((NP, F), jnp.float32),
    )(h, wroot, b)


def _k1_body(p0_ref, p1_ref, r_ref, wrel2_ref, h1_ref, y2_ref):
    h1 = jnp.maximum(p0_ref[0] + p1_ref[0] + r_ref[...], 0.0)
    h1_ref[...] = h1
    y2_ref[...] = jnp.dot(h1, wrel2_ref[...],
                          preferred_element_type=jnp.float32)


def _combine_and_next(part, r, wrel2):
    return pl.pallas_call(
        _k1_body,
        grid=(_GRID,),
        in_specs=[_part_spec(0), _part_spec(1), _row_spec(),
                  _full_spec((F, F))],
        out_specs=[_row_spec(), _row_spec()],
        out_shape=[jax.ShapeDtypeStruct((NP, F), jnp.float32),
                   jax.ShapeDtypeStruct((NP, F), jnp.float32)],
    )(part, part, r, wrel2)


def _k2_body(p0_ref, p1_ref, r_ref, pcol_ref, h2_ref, q_ref):
    h2 = jnp.maximum(p0_ref[0] + p1_ref[0] + r_ref[...], 0.0)
    h2_ref[...] = h2
    q_ref[...] = jnp.dot(h2, pcol_ref[...],
                         preferred_element_type=jnp.float32)


def _combine_and_score(part, r, pcol):
    return pl.pallas_call(
        _k2_body,
        grid=(_GRID,),
        in_specs=[_part_spec(0), _part_spec(1), _row_spec(),
                  _full_spec((F, 1))],
        out_specs=[_row_spec(), pl.BlockSpec((_BLK, 1), lambda i: (i, 0))],
        out_shape=[jax.ShapeDtypeStruct((NP, F), jnp.float32),
                   jax.ShapeDtypeStruct((NP, 1), jnp.float32)],
    )(part, part, r, pcol)


def _pool_body(q_ref, batch_ref, pp_ref, s_ref, keep_ref, counts_ref):
    q = q_ref[...]
    batch = batch_ref[...]
    bits = lax.bitcast_convert_type(q, jnp.int32)
    key = bits ^ (lax.shift_right_arithmetic(bits, 31) & jnp.int32(0x7FFFFFFF))

    mg = [batch == g for g in range(N_GRAPHS)]
    mgf = [m.astype(jnp.float32) for m in mg]
    ones_col = jnp.ones((128, 1), jnp.float32)

    def select_chain(scalars):
        t = jnp.broadcast_to(scalars[N_GRAPHS - 1], q.shape)
        for g in range(N_GRAPHS - 1):
            t = jnp.where(mg[g], scalars[g], t)
        return t

    def gcounts(predf, masksf):
        rows = [jnp.sum(predf * masksf[g], axis=0, keepdims=True)
                for g in range(N_GRAPHS)]
        return lax.dot_general(jnp.concatenate(rows, axis=0), ones_col,
                               (((1,), (0,)), ((), ())),
                               preferred_element_type=jnp.float32)

    onesf = jnp.ones(q.shape, jnp.float32)
    sizes = gcounts(onesf, mgf)
    kper = jnp.ceil(jnp.float32(0.8) * sizes)
    kper_i = kper.astype(jnp.int32)

    def t_body(i, tu):
        b = 31 - i
        bit = lax.shift_left(jnp.int32(1), b)
        cand_u = tu | bit
        cand_k = cand_u ^ jnp.int32(INT_MIN32)
        tn = select_chain([cand_k[g, 0] for g in range(N_GRAPHS)])
        predf = (key >= tn).astype(jnp.float32)
        cnt = gcounts(predf, mgf)
        return jnp.where(cnt >= kper, cand_u, tu)

    tu = lax.fori_loop(0, 32, t_body, jnp.zeros((N_GRAPHS, 1), jnp.int32))
    tk = tu ^ jnp.int32(INT_MIN32)

    tkn = select_chain([tk[g, 0] for g in range(N_GRAPHS)])
    tief = (key == tkn).astype(jnp.float32)
    wgf = [tief * mgf[g] for g in range(N_GRAPHS)]
    cgt = gcounts((key > tkn).astype(jnp.float32), mgf)
    m = kper - cgt

    idx = (lax.broadcasted_iota(jnp.int32, q.shape, 0) * 128
           + lax.broadcasted_iota(jnp.int32, q.shape, 1))

    def j_body(i, jv):
        b = 13 - i
        bit = lax.shift_left(jnp.int32(1), b)
        cand = jv | bit
        jn = select_chain([cand[g, 0] for g in range(N_GRAPHS)])
        cnt = gcounts((idx < jn).astype(jnp.float32), wgf)
        return jnp.where(cnt < m, cand, jv)

    jv = lax.fori_loop(0, 14, j_body, jnp.zeros((N_GRAPHS, 1), jnp.int32))

    jn = select_chain([jv[g, 0] for g in range(N_GRAPHS)])
    keep = (key > tkn) | ((key == tkn) & (idx <= jn))
    keep = keep & (batch < N_GRAPHS)
    keep_f = keep.astype(jnp.float32)

    pp = pp_ref[...]
    inv_norm = lax.rsqrt(jnp.sum(pp * pp))
    s_ref[...] = jnp.tanh(q * inv_norm) * keep_f
    keep_ref[...] = keep_f
    counts_ref[...] = gcounts(keep_f, mgf)


def _pool_select(q2, batch2, pp):
    return pl.pallas_call(
        _pool_body,
        grid=(1,),
        in_specs=[_full_spec((NROWS, 128)), _full_spec((NROWS, 128)),
                  _full_spec((1, F))],
        out_specs=[_full_spec((NROWS, 128)), _full_spec((NROWS, 128)),
                   _full_spec((N_GRAPHS, 1))],
        out_shape=[jax.ShapeDtypeStruct((NROWS, 128), jnp.float32),
                   jax.ShapeDtypeStruct((NROWS, 128), jnp.float32),
                   jax.ShapeDtypeStruct((N_GRAPHS, 1), jnp.float32)],
    )(q2, batch2, pp)


def _k3b_body(h2_ref, s_ref, wrel3_ref, h2p_ref, y3_ref):
    h2p = h2_ref[...] * s_ref[...]
    h2p_ref[...] = h2p
    y3_ref[...] = jnp.dot(h2p, wrel3_ref[...],
                          preferred_element_type=jnp.float32)


def _scale_and_next(h2, s, wrel3):
    return pl.pallas_call(
        _k3b_body,
        grid=(_GRID,),
        in_specs=[_row_spec(), pl.BlockSpec((_BLK, 1), lambda i: (i, 0)),
                  _full_spec((F, F))],
        out_specs=[_row_spec(), _row_spec()],
        out_shape=[jax.ShapeDtypeStruct((NP, F), jnp.float32),
                   jax.ShapeDtypeStruct((NP, F), jnp.float32)],
    )(h2, s, wrel3)


def _k45_body(p0_ref, p1_ref, r_ref, keep_ref,
              batch_ref, counts_ref, w1_ref, b1_ref, w2_ref, b2_ref,
              o_ref, gsum_ref):
    @pl.when(pl.program_id(0) == 0)
    def _init():
        gsum_ref[...] = jnp.zeros_like(gsum_ref)

    h3 = keep_ref[...] * jnp.maximum(
        p0_ref[0] + p1_ref[0] + r_ref[...], 0.0)
    oh = (batch_ref[...] == lax.broadcasted_iota(
        jnp.int32, (_BLK, N_GRAPHS), 1)).astype(jnp.float32)
    gsum_ref[...] += lax.dot_general(
        oh, h3, (((0,), (0,)), ((), ())),
        preferred_element_type=jnp.float32)

    @pl.when(pl.program_id(0) == _GRID - 1)
    def _head():
        mean = gsum_ref[...] / jnp.maximum(counts_ref[...], 1.0)
        z = jnp.maximum(jnp.dot(mean, w1_ref[...],
                                preferred_element_type=jnp.float32)
                        + b1_ref[...], 0.0)
        logits = jnp.dot(z, w2_ref[...],
                         preferred_element_type=jnp.float32) + b2_ref[...]
        mx = jnp.max(logits, axis=1, keepdims=True)
        lse = jnp.log(jnp.sum(jnp.exp(logits - mx), axis=1, keepdims=True))
        o_ref[...] = logits - mx - lse


def _final_conv_pool_head(part, r, keep, batch_col,
                          counts, w1, b1, w2, b2):
    return pl.pallas_call(
        _k45_body,
        grid=(_GRID,),
        in_specs=[_part_spec(0), _part_spec(1), _row_spec(),
                  pl.BlockSpec((_BLK, 1), lambda i: (i, 0)),
                  pl.BlockSpec((_BLK, 1), lambda i: (i, 0)),
                  _full_spec((N_GRAPHS, 1)),
                  _full_spec((F, 64)), _full_spec((1, 64)),
                  _full_spec((64, 10)), _full_spec((1, 10))],
        out_specs=_full_spec((N_GRAPHS, 10)),
        out_shape=jax.ShapeDtypeStruct((N_GRAPHS, 10), jnp.float32),
        scratch_shapes=[pltpu.VMEM((N_GRAPHS, F), jnp.float32)],
    )(part, part, r, keep, batch_col, counts, w1, b1, w2, b2)


def kernel(x, edge_index, batch,
           conv1_Wrel, conv1_brel, conv1_Wroot,
           conv2_Wrel, conv2_brel, conv2_Wroot,
           conv3_Wrel, conv3_brel, conv3_Wroot,
           pool_p, lin1_W, lin1_b, lin2_W, lin2_b):
    n = x.shape[0]
    xp = jnp.pad(x, ((0, NP - n), (0, 0)))
    src = edge_index[0].astype(jnp.int32)
    dst = edge_index[1].astype(jnp.int32)
    nfe = NFULL * CHUNK * NC * NS
    src2d = src[:nfe].reshape(NC * NS, NFULL, CHUNK)
    dst2d = dst[:nfe].reshape(NC * NS, NFULL, CHUNK)
    srct = src[nfe:]
    dstt = dst[nfe:]
    batch_pad = jnp.pad(batch.astype(jnp.int32), (0, NP - n),
                        constant_values=127)
    batch2 = batch_pad.reshape(NROWS, 128)
    batch_col = batch_pad.reshape(NP, 1)
    b1 = conv1_brel.reshape(1, F)
    b2 = conv2_brel.reshape(1, F)
    b3 = conv3_brel.reshape(1, F)
    pp_row = pool_p.reshape(1, F)
    pcol = pool_p.reshape(F, 1)

    y1 = _matmul(xp, conv1_Wrel)
    part1 = _sc_segment_sum(y1, src2d, dst2d, srct, dstt)
    r1 = _root(xp, conv1_Wroot, b1)
    h1, y2 = _combine_and_next(part1, r1, conv2_Wrel)

    part2 = _sc_segment_sum(y2, src2d, dst2d, srct, dstt)
    r2 = _root(h1, conv2_Wroot, b2)
    h2, q = _combine_and_score(part2, r2, pcol)

    s2, keep2, counts = _pool_select(q.reshape(NROWS, 128), batch2, pp_row)
    s_col = s2.reshape(NP, 1)
    keep_col = keep2.reshape(NP, 1)

    h2p, y3 = _scale_and_next(h2, s_col, conv3_Wrel)
    part3 = _sc_segment_sum(y3, src2d, dst2d, srct, dstt)
    r3 = _root(h2p, conv3_Wroot, b3)
    return _final_conv_pool_head(part3, r3,
                                 keep_col, batch_col, counts,
                                 lin1_W, lin1_b.reshape(1, 64),
                                 lin2_W, lin2_b.reshape(1, 10))


# speedup vs baseline: 1.1785x; 1.0561x over previous
import functools

import jax
import jax.numpy as jnp
from jax import lax
from jax.experimental import pallas as pl
from jax.experimental.pallas import tpu as pltpu
from jax.experimental.pallas import tpu_sc as plsc

N_GRAPHS = 16
NP = 10240
NROWS = NP // 128
F = 128
E = 320000
NC, NS = 2, 16
EDGES_PER_TILE = E // (NC * NS)
CHUNK = 128
NFULL = EDGES_PER_TILE // CHUNK
REM = EDGES_PER_TILE - NFULL * CHUNK
ROWS_PER_TILE = NP // NS
STAGE = 24
INT_MIN32 = -2147483648


def _sc_segment_sum(y, src2d, dst2d, srct, dstt):
    mesh = plsc.VectorSubcoreMesh(
        core_axis_name="c", subcore_axis_name="s", num_cores=NC,
        num_subcores=NS)

    @functools.partial(
        pl.kernel,
        out_type=jax.ShapeDtypeStruct((NC, NP, F), jnp.float32),
        mesh=mesh,
        scratch_types=[
            pltpu.VMEM((STAGE, CHUNK), jnp.int32),
            pltpu.VMEM((STAGE, CHUNK), jnp.int32),
            pltpu.VMEM((CHUNK, F), jnp.float32),
            pltpu.VMEM((CHUNK, F), jnp.float32),
            pltpu.VMEM((REM,), jnp.int32),
            pltpu.VMEM((REM,), jnp.int32),
            pltpu.VMEM((16, F), jnp.float32),
            pltpu.VMEM_SHARED((NP, F), jnp.float32),
            pltpu.SemaphoreType.DMA,
            pltpu.SemaphoreType.DMA,
            pltpu.SemaphoreType.DMA,
            pltpu.SemaphoreType.DMA,
            pltpu.SemaphoreType.DMA,
        ],
    )
    def k(y_hbm, src_hbm, dst_hbm, srct_hbm, dstt_hbm, out_hbm,
          sidx, didx, rows0, rows1, sidxr, didxr, zbuf, acc,
          sg0, sg1, ss0, ss1, semi):
        c = lax.axis_index("c")
        s = lax.axis_index("s")
        w = c * NS + s

        i0 = pltpu.make_async_copy(src_hbm.at[w, pl.ds(0, STAGE)], sidx, semi)
        i1 = pltpu.make_async_copy(dst_hbm.at[w, pl.ds(0, STAGE)], didx, semi)
        i2 = pltpu.make_async_copy(srct_hbm.at[pl.ds(w * REM, REM)],
                                   sidxr, semi)
        i3 = pltpu.make_async_copy(dstt_hbm.at[pl.ds(w * REM, REM)],
                                   didxr, semi)
        i0.start(); i1.start(); i2.start(); i3.start()

        for i in range(16):
            for j in range(F // 16):
                zbuf[i, pl.ds(j * 16, 16)] = jnp.zeros((16,), jnp.float32)
        row0 = s * ROWS_PER_TILE

        def zero_body(kk, _):
            pltpu.sync_copy(zbuf, acc.at[pl.ds(row0 + kk * 16, 16)])
            return _
        lax.fori_loop(0, ROWS_PER_TILE // 16, zero_body, 0)
        i0.wait(); i1.wait(); i2.wait(); i3.wait()
        plsc.subcore_barrier()

        def gather(kk, buf, sg):
            return pltpu.make_async_copy(y_hbm.at[sidx.at[kk]], buf, sg)

        def run_stage(npairs):
            gather(0, rows0, sg0).start()

            def pair_body(kk, _):
                e = 2 * kk
                gather(e, rows0, sg0).wait()
                gather(e + 1, rows1, sg1).start()
                pltpu.sync_copy(rows0, acc.at[didx.at[e]], add=True)
                gather(e + 1, rows1, sg1).wait()

                @pl.when(kk < npairs - 1)
                def _next():
                    gather(e + 2, rows0, sg0).start()

                pltpu.sync_copy(rows1, acc.at[didx.at[e + 1]], add=True)
                return _
            lax.fori_loop(0, npairs, pair_body, 0)

        for t in range(NFULL // STAGE):
            if t > 0:
                pltpu.sync_copy(src_hbm.at[w, pl.ds(t * STAGE, STAGE)], sidx)
                pltpu.sync_copy(dst_hbm.at[w, pl.ds(t * STAGE, STAGE)], didx)
            run_stage(STAGE // 2)
        tcount = NFULL - (NFULL // STAGE) * STAGE
        pltpu.sync_copy(
            src_hbm.at[w, pl.ds((NFULL // STAGE) * STAGE, tcount)],
            sidx.at[pl.ds(0, tcount)])
        pltpu.sync_copy(
            dst_hbm.at[w, pl.ds((NFULL // STAGE) * STAGE, tcount)],
            didx.at[pl.ds(0, tcount)])
        run_stage(tcount // 2)

        pltpu.async_copy(y_hbm.at[sidxr], rows1.at[pl.ds(0, REM)], sg1).wait()
        pltpu.sync_copy(rows1.at[pl.ds(0, REM)], acc.at[didxr], add=True)

        plsc.subcore_barrier()
        pltpu.sync_copy(acc.at[pl.ds(row0, ROWS_PER_TILE)],
                        out_hbm.at[c].at[pl.ds(row0, ROWS_PER_TILE)])

    return k(y, src2d, dst2d, srct, dstt)


_BLK = 1024
_GRID = NP // _BLK


def _row_spec(blk=_BLK, width=F):
    return pl.BlockSpec((blk, width), lambda i: (i, 0))


def _part_spec(core):
    return pl.BlockSpec((1, _BLK, F), lambda i, c=core: (c, i, 0))


def _full_spec(shape):
    return pl.BlockSpec(shape, lambda i: tuple(0 for _ in shape))


def _k0_body(x_ref, w_ref, o_ref):
    o_ref[...] = jnp.dot(x_ref[...], w_ref[...],
                         preferred_element_type=jnp.float32)


def _matmul(x, w):
    return pl.pallas_call(
        _k0_body,
        grid=(_GRID,),
        in_specs=[_row_spec(), _full_spec((F, F))],
        out_specs=_row_spec(),
        out_shape=jax.ShapeDtypeStruct((NP, F), jnp.float32),
    )(x, w)


def _kr_body(h_ref, wroot_ref, b_ref, r_ref):
    r_ref[...] = jnp.dot(h_ref[...], wroot_ref[...],
                         preferred_element_type=jnp.float32) + b_ref[...]


def _root(h, wroot, b):
    return pl.pallas_call(
        _kr_body,
        grid=(_GRID,),
        in_specs=[_row_spec(), _full_spec((F, F)), _full_spec((1, F))],
        out_specs=_row_spec(),
        out_shape=jax.ShapeDtypeStruct((NP, F), jnp.float32),
    )(h, wroot, b)


def _k1_body(p0_ref, p1_ref, r_ref, wrel2_ref, h1_ref, y2_ref):
    h1 = jnp.maximum(p0_ref[0] + p1_ref[0] + r_ref[...], 0.0)
    h1_ref[...] = h1
    y2_ref[...] = jnp.dot(h1, wrel2_ref[...],
                          preferred_element_type=jnp.float32)


def _combine_and_next(part, r, wrel2):
    return pl.pallas_call(
        _k1_body,
        grid=(_GRID,),
        in_specs=[_part_spec(0), _part_spec(1), _row_spec(),
                  _full_spec((F, F))],
        out_specs=[_row_spec(), _row_spec()],
        out_shape=[jax.ShapeDtypeStruct((NP, F), jnp.float32),
                   jax.ShapeDtypeStruct((NP, F), jnp.float32)],
    )(part, part, r, wrel2)


def _k2_body(p0_ref, p1_ref, r_ref, pcol_ref, h2_ref, q_ref):
    h2 = jnp.maximum(p0_ref[0] + p1_ref[0] + r_ref[...], 0.0)
    h2_ref[...] = h2
    q_ref[...] = jnp.dot(h2, pcol_ref[...],
                         preferred_element_type=jnp.float32)


def _combine_and_score(part, r, pcol):
    return pl.pallas_call(
        _k2_body,
        grid=(_GRID,),
        in_specs=[_part_spec(0), _part_spec(1), _row_spec(),
                  _full_spec((F, 1))],
        out_specs=[_row_spec(), pl.BlockSpec((_BLK, 1), lambda i: (i, 0))],
        out_shape=[jax.ShapeDtypeStruct((NP, F), jnp.float32),
                   jax.ShapeDtypeStruct((NP, 1), jnp.float32)],
    )(part, part, r, pcol)


def _pool_body(q_ref, batch_ref, pp_ref, s_ref, keep_ref, counts_ref):
    q = q_ref[...]
    batch = batch_ref[...]
    bits = lax.bitcast_convert_type(q, jnp.int32)
    key = bits ^ (lax.shift_right_arithmetic(bits, 31) & jnp.int32(0x7FFFFFFF))

    mg = [batch == g for g in range(N_GRAPHS)]
    mgf = [m.astype(jnp.float32) for m in mg]
    ones_col = jnp.ones((128, 1), jnp.float32)

    def select_chain(scalars):
        t = jnp.broadcast_to(scalars[N_GRAPHS - 1], q.shape)
        for g in range(N_GRAPHS - 1):
            t = jnp.where(mg[g], scalars[g], t)
        return t

    def gcounts(predf, masksf):
        rows = [jnp.sum(predf * masksf[g], axis=0, keepdims=True)
                for g in range(N_GRAPHS)]
        return lax.dot_general(jnp.concatenate(rows, axis=0), ones_col,
                               (((1,), (0,)), ((), ())),
                               preferred_element_type=jnp.float32)

    onesf = jnp.ones(q.shape, jnp.float32)
    sizes = gcounts(onesf, mgf)
    kper = jnp.ceil(jnp.float32(0.8) * sizes)
    kper_i = kper.astype(jnp.int32)

    def t_body(i, tu):
        b = 31 - i
        bit = lax.shift_left(jnp.int32(1), b)
        cand_u = tu | bit
        cand_k = cand_u ^ jnp.int32(INT_MIN32)
        tn = select_chain([cand_k[g, 0] for g in range(N_GRAPHS)])
        predf = (key >= tn).astype(jnp.float32)
        cnt = gcounts(predf, mgf)
        return jnp.where(cnt >= kper, cand_u, tu)

    tu = lax.fori_loop(0, 32, t_body, jnp.zeros((N_GRAPHS, 1), jnp.int32))
    tk = tu ^ jnp.int32(INT_MIN32)

    tkn = select_chain([tk[g, 0] for g in range(N_GRAPHS)])
    tief = (key == tkn).astype(jnp.float32)
    wgf = [tief * mgf[g] for g in range(N_GRAPHS)]
    cgt = gcounts((key > tkn).astype(jnp.float32), mgf)
    m = kper - cgt

    idx = (lax.broadcasted_iota(jnp.int32, q.shape, 0) * 128
           + lax.broadcasted_iota(jnp.int32, q.shape, 1))

    def j_body(i, jv):
        b = 13 - i
        bit = lax.shift_left(jnp.int32(1), b)
        cand = jv | bit
        jn = select_chain([cand[g, 0] for g in range(N_GRAPHS)])
        cnt = gcounts((idx < jn).astype(jnp.float32), wgf)
        return jnp.where(cnt < m, cand, jv)

    jv = lax.fori_loop(0, 14, j_body, jnp.zeros((N_GRAPHS, 1), jnp.int32))

    jn = select_chain([jv[g, 0] for g in range(N_GRAPHS)])
    keep = (key > tkn) | ((key == tkn) & (idx <= jn))
    keep = keep & (batch < N_GRAPHS)
    keep_f = keep.astype(jnp.float32)

    pp = pp_ref[...]
    inv_norm = lax.rsqrt(jnp.sum(pp * pp))
    s_ref[...] = jnp.tanh(q * inv_norm) * keep_f
    keep_ref[...] = keep_f
    counts_ref[...] = gcounts(keep_f, mgf)


def _pool_select(q2, batch2, pp):
    return pl.pallas_call(
        _pool_body,
        grid=(1,),
        in_specs=[_full_spec((NROWS, 128)), _full_spec((NROWS, 128)),
                  _full_spec((1, F))],
        out_specs=[_full_spec((NROWS, 128)), _full_spec((NROWS, 128)),
                   _full_spec((N_GRAPHS, 1))],
        out_shape=[jax.ShapeDtypeStruct((NROWS, 128), jnp.float32),
                   jax.ShapeDtypeStruct((NROWS, 128), jnp.float32),
                   jax.ShapeDtypeStruct((N_GRAPHS, 1), jnp.float32)],
    )(q2, batch2, pp)


def _k3b_body(h2_ref, s_ref, wrel3_ref, h2p_ref, y3_ref):
    h2p = h2_ref[...] * s_ref[...]
    h2p_ref[...] = h2p
    y3_ref[...] = jnp.dot(h2p, wrel3_ref[...],
                          preferred_element_type=jnp.float32)


def _scale_and_next(h2, s, wrel3):
    return pl.pallas_call(
        _k3b_body,
        grid=(_GRID,),
        in_specs=[_row_spec(), pl.BlockSpec((_BLK, 1), lambda i: (i, 0)),
                  _full_spec((F, F))],
        out_specs=[_row_spec(), _row_spec()],
        out_shape=[jax.ShapeDtypeStruct((NP, F), jnp.float32),
                   jax.ShapeDtypeStruct((NP, F), jnp.float32)],
    )(h2, s, wrel3)


def _k45_body(p0_ref, p1_ref, r_ref, keep_ref,
              batch_ref, counts_ref, w1_ref, b1_ref, w2_ref, b2_ref,
              o_ref, gsum_ref):
    @pl.when(pl.program_id(0) == 0)
    def _init():
        gsum_ref[...] = jnp.zeros_like(gsum_ref)

    h3 = keep_ref[...] * jnp.maximum(
        p0_ref[0] + p1_ref[0] + r_ref[...], 0.0)
    oh = (batch_ref[...] == lax.broadcasted_iota(
        jnp.int32, (_BLK, N_GRAPHS), 1)).astype(jnp.float32)
    gsum_ref[...] += lax.dot_general(
        oh, h3, (((0,), (0,)), ((), ())),
        preferred_element_type=jnp.float32)

    @pl.when(pl.program_id(0) == _GRID - 1)
    def _head():
        mean = gsum_ref[...] / jnp.maximum(counts_ref[...], 1.0)
        z = jnp.maximum(jnp.dot(mean, w1_ref[...],
                                preferred_element_type=jnp.float32)
                        + b1_ref[...], 0.0)
        logits = jnp.dot(z, w2_ref[...],
                         preferred_element_type=jnp.float32) + b2_ref[...]
        mx = jnp.max(logits, axis=1, keepdims=True)
        lse = jnp.log(jnp.sum(jnp.exp(logits - mx), axis=1, keepdims=True))
        o_ref[...] = logits - mx - lse


def _final_conv_pool_head(part, r, keep, batch_col,
                          counts, w1, b1, w2, b2):
    return pl.pallas_call(
        _k45_body,
        grid=(_GRID,),
        in_specs=[_part_spec(0), _part_spec(1), _row_spec(),
                  pl.BlockSpec((_BLK, 1), lambda i: (i, 0)),
                  pl.BlockSpec((_BLK, 1), lambda i: (i, 0)),
                  _full_spec((N_GRAPHS, 1)),
                  _full_spec((F, 64)), _full_spec((1, 64)),
                  _full_spec((64, 10)), _full_spec((1, 10))],
        out_specs=_full_spec((N_GRAPHS, 10)),
        out_shape=jax.ShapeDtypeStruct((N_GRAPHS, 10), jnp.float32),
        scratch_shapes=[pltpu.VMEM((N_GRAPHS, F), jnp.float32)],
    )(part, part, r, keep, batch_col, counts, w1, b1, w2, b2)


def kernel(x, edge_index, batch,
           conv1_Wrel, conv1_brel, conv1_Wroot,
           conv2_Wrel, conv2_brel, conv2_Wroot,
           conv3_Wrel, conv3_brel, conv3_Wroot,
           pool_p, lin1_W, lin1_b, lin2_W, lin2_b):
    n = x.shape[0]
    xp = jnp.pad(x, ((0, NP - n), (0, 0)))
    src = edge_index[0].astype(jnp.int32)
    dst = edge_index[1].astype(jnp.int32)
    nfe = NFULL * CHUNK * NC * NS
    src2d = src[:nfe].reshape(NC * NS, NFULL, CHUNK)
    dst2d = dst[:nfe].reshape(NC * NS, NFULL, CHUNK)
    srct = src[nfe:]
    dstt = dst[nfe:]
    batch_pad = jnp.pad(batch.astype(jnp.int32), (0, NP - n),
                        constant_values=127)
    batch2 = batch_pad.reshape(NROWS, 128)
    batch_col = batch_pad.reshape(NP, 1)
    b1 = conv1_brel.reshape(1, F)
    b2 = conv2_brel.reshape(1, F)
    b3 = conv3_brel.reshape(1, F)
    pp_row = pool_p.reshape(1, F)
    pcol = pool_p.reshape(F, 1)

    y1 = _matmul(xp, conv1_Wrel)
    part1 = _sc_segment_sum(y1, src2d, dst2d, srct, dstt)
    r1 = _root(xp, conv1_Wroot, b1)
    h1, y2 = _combine_and_next(part1, r1, conv2_Wrel)

    part2 = _sc_segment_sum(y2, src2d, dst2d, srct, dstt)
    r2 = _root(h1, conv2_Wroot, b2)
    h2, q = _combine_and_score(part2, r2, pcol)

    s2, keep2, counts = _pool_select(q.reshape(NROWS, 128), batch2, pp_row)
    s_col = s2.reshape(NP, 1)
    keep_col = keep2.reshape(NP, 1)

    h2p, y3 = _scale_and_next(h2, s_col, conv3_Wrel)
    part3 = _sc_segment_sum(y3, src2d, dst2d, srct, dstt)
    r3 = _root(h2p, conv3_Wroot, b3)
    return _final_conv_pool_head(part3, r3,
                                 keep_col, batch_col, counts,
                                 lin1_W, lin1_b.reshape(1, 64),
                                 lin2_W, lin2_b.reshape(1, 10))


# speedup vs baseline: 1.1983x; 1.0168x over previous
import functools

import jax
import jax.numpy as jnp
from jax import lax
from jax.experimental import pallas as pl
from jax.experimental.pallas import tpu as pltpu
from jax.experimental.pallas import tpu_sc as plsc

N_GRAPHS = 16
NP = 10240
NROWS = NP // 128
F = 128
E = 320000
NC, NS = 2, 16
EDGES_PER_TILE = E // (NC * NS)
CHUNK = 128
NFULL = EDGES_PER_TILE // CHUNK
REM = EDGES_PER_TILE - NFULL * CHUNK
ROWS_PER_TILE = NP // NS
STAGE = 24
INT_MIN32 = -2147483648


def _sc_segment_sum(y, src2d, dst2d, srct, dstt):
    mesh = plsc.VectorSubcoreMesh(
        core_axis_name="c", subcore_axis_name="s", num_cores=NC,
        num_subcores=NS)

    @functools.partial(
        pl.kernel,
        out_type=jax.ShapeDtypeStruct((NC, NP, F), jnp.float32),
        mesh=mesh,
        scratch_types=[
            pltpu.VMEM((STAGE, CHUNK), jnp.int32),
            pltpu.VMEM((STAGE, CHUNK), jnp.int32),
            pltpu.VMEM((CHUNK, F), jnp.float32),
            pltpu.VMEM((CHUNK, F), jnp.float32),
            pltpu.VMEM((REM,), jnp.int32),
            pltpu.VMEM((REM,), jnp.int32),
            pltpu.VMEM((16, F), jnp.float32),
            pltpu.VMEM_SHARED((NP, F), jnp.float32),
            pltpu.SemaphoreType.DMA,
            pltpu.SemaphoreType.DMA,
            pltpu.SemaphoreType.DMA,
            pltpu.SemaphoreType.DMA,
            pltpu.SemaphoreType.DMA,
        ],
    )
    def k(y_hbm, src_hbm, dst_hbm, srct_hbm, dstt_hbm, out_hbm,
          sidx, didx, rows0, rows1, sidxr, didxr, zbuf, acc,
          sg0, sg1, ss0, ss1, semi):
        c = lax.axis_index("c")
        s = lax.axis_index("s")
        w = c * NS + s

        i0 = pltpu.make_async_copy(src_hbm.at[w, pl.ds(0, STAGE)], sidx, semi)
        i1 = pltpu.make_async_copy(dst_hbm.at[w, pl.ds(0, STAGE)], didx, semi)
        i2 = pltpu.make_async_copy(srct_hbm.at[pl.ds(w * REM, REM)],
                                   sidxr, semi)
        i3 = pltpu.make_async_copy(dstt_hbm.at[pl.ds(w * REM, REM)],
                                   didxr, semi)
        i0.start(); i1.start(); i2.start(); i3.start()

        for i in range(16):
            for j in range(F // 16):
                zbuf[i, pl.ds(j * 16, 16)] = jnp.zeros((16,), jnp.float32)
        row0 = s * ROWS_PER_TILE

        def zero_body(kk, _):
            pltpu.sync_copy(zbuf, acc.at[pl.ds(row0 + kk * 16, 16)])
            return _
        lax.fori_loop(0, ROWS_PER_TILE // 16, zero_body, 0)
        i0.wait(); i1.wait(); i2.wait(); i3.wait()
        plsc.subcore_barrier()

        def gather(kk, buf, sg):
            return pltpu.make_async_copy(y_hbm.at[sidx.at[kk]], buf, sg)

        def run_stage(npairs):
            gather(0, rows0, sg0).start()

            def pair_body(kk, _):
                e = 2 * kk
                gather(e, rows0, sg0).wait()
                gather(e + 1, rows1, sg1).start()
                pltpu.sync_copy(rows0, acc.at[didx.at[e]], add=True)
                gather(e + 1, rows1, sg1).wait()

                @pl.when(kk < npairs - 1)
                def _next():
                    gather(e + 2, rows0, sg0).start()

                pltpu.sync_copy(rows1, acc.at[didx.at[e + 1]], add=True)
                return _
            lax.fori_loop(0, npairs, pair_body, 0)

        for t in range(NFULL // STAGE):
            if t > 0:
                pltpu.sync_copy(src_hbm.at[w, pl.ds(t * STAGE, STAGE)], sidx)
                pltpu.sync_copy(dst_hbm.at[w, pl.ds(t * STAGE, STAGE)], didx)
            run_stage(STAGE // 2)
        tcount = NFULL - (NFULL // STAGE) * STAGE
        pltpu.sync_copy(
            src_hbm.at[w, pl.ds((NFULL // STAGE) * STAGE, tcount)],
            sidx.at[pl.ds(0, tcount)])
        pltpu.sync_copy(
            dst_hbm.at[w, pl.ds((NFULL // STAGE) * STAGE, tcount)],
            didx.at[pl.ds(0, tcount)])
        run_stage(tcount // 2)

        pltpu.async_copy(y_hbm.at[sidxr], rows1.at[pl.ds(0, REM)], sg1).wait()
        pltpu.sync_copy(rows1.at[pl.ds(0, REM)], acc.at[didxr], add=True)

        plsc.subcore_barrier()
        pltpu.sync_copy(acc.at[pl.ds(row0, ROWS_PER_TILE)],
                        out_hbm.at[c].at[pl.ds(row0, ROWS_PER_TILE)])

    return k(y, src2d, dst2d, srct, dstt)


_BLK = 2048
_GRID = NP // _BLK


def _row_spec(blk=_BLK, width=F):
    return pl.BlockSpec((blk, width), lambda i: (i, 0))


def _part_spec(core):
    return pl.BlockSpec((1, _BLK, F), lambda i, c=core: (c, i, 0))


def _full_spec(shape):
    return pl.BlockSpec(shape, lambda i: tuple(0 for _ in shape))


def _k0_body(x_ref, w_ref, o_ref):
    o_ref[...] = jnp.dot(x_ref[...], w_ref[...],
                         preferred_element_type=jnp.float32)


def _matmul(x, w):
    return pl.pallas_call(
        _k0_body,
        grid=(_GRID,),
        in_specs=[_row_spec(), _full_spec((F, F))],
        out_specs=_row_spec(),
        out_shape=jax.ShapeDtypeStruct((NP, F), jnp.float32),
    )(x, w)


def _kr_body(h_ref, wroot_ref, b_ref, r_ref):
    r_ref[...] = jnp.dot(h_ref[...], wroot_ref[...],
                         preferred_element_type=jnp.float32) + b_ref[...]


def _root(h, wroot, b):
    return pl.pallas_call(
        _kr_body,
        grid=(_GRID,),
        in_specs=[_row_spec(), _full_spec((F, F)), _full_spec((1, F))],
        out_specs=_row_spec(),
        out_shape=jax.ShapeDtypeStruct((NP, F), jnp.float32),
    )(h, wroot, b)


def _k1_body(p0_ref, p1_ref, r_ref, wrel2_ref, h1_ref, y2_ref):
    h1 = jnp.maximum(p0_ref[0] + p1_ref[0] + r_ref[...], 0.0)
    h1_ref[...] = h1
    y2_ref[...] = jnp.dot(h1, wrel2_ref[...],
                          preferred_element_type=jnp.float32)


def _combine_and_next(part, r, wrel2):
    return pl.pallas_call(
        _k1_body,
        grid=(_GRID,),
        in_specs=[_part_spec(0), _part_spec(1), _row_spec(),
                  _full_spec((F, F))],
        out_specs=[_row_spec(), _row_spec()],
        out_shape=[jax.ShapeDtypeStruct((NP, F), jnp.float32),
                   jax.ShapeDtypeStruct((NP, F), jnp.float32)],
    )(part, part, r, wrel2)


def _k2_body(p0_ref, p1_ref, r_ref, pcol_ref, h2_ref, q_ref):
    h2 = jnp.maximum(p0_ref[0] + p1_ref[0] + r_ref[...], 0.0)
    h2_ref[...] = h2
    q_ref[...] = jnp.dot(h2, pcol_ref[...],
                         preferred_element_type=jnp.float32)


def _combine_and_score(part, r, pcol):
    return pl.pallas_call(
        _k2_body,
        grid=(_GRID,),
        in_specs=[_part_spec(0), _part_spec(1), _row_spec(),
                  _full_spec((F, 1))],
        out_specs=[_row_spec(), pl.BlockSpec((_BLK, 1), lambda i: (i, 0))],
        out_shape=[jax.ShapeDtypeStruct((NP, F), jnp.float32),
                   jax.ShapeDtypeStruct((NP, 1), jnp.float32)],
    )(part, part, r, pcol)


def _pool_body(q_ref, batch_ref, pp_ref, s_ref, keep_ref, counts_ref):
    q = q_ref[...]
    batch = batch_ref[...]
    bits = lax.bitcast_convert_type(q, jnp.int32)
    key = bits ^ (lax.shift_right_arithmetic(bits, 31) & jnp.int32(0x7FFFFFFF))

    mg = [batch == g for g in range(N_GRAPHS)]
    mgf = [m.astype(jnp.float32) for m in mg]
    ones_col = jnp.ones((128, 1), jnp.float32)

    def select_chain(scalars):
        t = jnp.broadcast_to(scalars[N_GRAPHS - 1], q.shape)
        for g in range(N_GRAPHS - 1):
            t = jnp.where(mg[g], scalars[g], t)
        return t

    def gcounts(predf, masksf):
        rows = [jnp.sum(predf * masksf[g], axis=0, keepdims=True)
                for g in range(N_GRAPHS)]
        return lax.dot_general(jnp.concatenate(rows, axis=0), ones_col,
                               (((1,), (0,)), ((), ())),
                               preferred_element_type=jnp.float32)

    onesf = jnp.ones(q.shape, jnp.float32)
    sizes = gcounts(onesf, mgf)
    kper = jnp.ceil(jnp.float32(0.8) * sizes)
    kper_i = kper.astype(jnp.int32)

    def t_body(i, tu):
        b = 31 - i
        bit = lax.shift_left(jnp.int32(1), b)
        cand_u = tu | bit
        cand_k = cand_u ^ jnp.int32(INT_MIN32)
        tn = select_chain([cand_k[g, 0] for g in range(N_GRAPHS)])
        predf = (key >= tn).astype(jnp.float32)
        cnt = gcounts(predf, mgf)
        return jnp.where(cnt >= kper, cand_u, tu)

    tu = lax.fori_loop(0, 32, t_body, jnp.zeros((N_GRAPHS, 1), jnp.int32))
    tk = tu ^ jnp.int32(INT_MIN32)

    tkn = select_chain([tk[g, 0] for g in range(N_GRAPHS)])
    tief = (key == tkn).astype(jnp.float32)
    wgf = [tief * mgf[g] for g in range(N_GRAPHS)]
    cgt = gcounts((key > tkn).astype(jnp.float32), mgf)
    m = kper - cgt

    idx = (lax.broadcasted_iota(jnp.int32, q.shape, 0) * 128
           + lax.broadcasted_iota(jnp.int32, q.shape, 1))

    def j_body(i, jv):
        b = 13 - i
        bit = lax.shift_left(jnp.int32(1), b)
        cand = jv | bit
        jn = select_chain([cand[g, 0] for g in range(N_GRAPHS)])
        cnt = gcounts((idx < jn).astype(jnp.float32), wgf)
        return jnp.where(cnt < m, cand, jv)

    jv = lax.fori_loop(0, 14, j_body, jnp.zeros((N_GRAPHS, 1), jnp.int32))

    jn = select_chain([jv[g, 0] for g in range(N_GRAPHS)])
    keep = (key > tkn) | ((key == tkn) & (idx <= jn))
    keep = keep & (batch < N_GRAPHS)
    keep_f = keep.astype(jnp.float32)

    pp = pp_ref[...]
    inv_norm = lax.rsqrt(jnp.sum(pp * pp))
    s_ref[...] = jnp.tanh(q * inv_norm) * keep_f
    keep_ref[...] = keep_f
    counts_ref[...] = gcounts(keep_f, mgf)


def _pool_select(q2, batch2, pp):
    return pl.pallas_call(
        _pool_body,
        grid=(1,),
        in_specs=[_full_spec((NROWS, 128)), _full_spec((NROWS, 128)),
                  _full_spec((1, F))],
        out_specs=[_full_spec((NROWS, 128)), _full_spec((NROWS, 128)),
                   _full_spec((N_GRAPHS, 1))],
        out_shape=[jax.ShapeDtypeStruct((NROWS, 128), jnp.float32),
                   jax.ShapeDtypeStruct((NROWS, 128), jnp.float32),
                   jax.ShapeDtypeStruct((N_GRAPHS, 1), jnp.float32)],
    )(q2, batch2, pp)


def _k3b_body(h2_ref, s_ref, wrel3_ref, h2p_ref, y3_ref):
    h2p = h2_ref[...] * s_ref[...]
    h2p_ref[...] = h2p
    y3_ref[...] = jnp.dot(h2p, wrel3_ref[...],
                          preferred_element_type=jnp.float32)


def _scale_and_next(h2, s, wrel3):
    return pl.pallas_call(
        _k3b_body,
        grid=(_GRID,),
        in_specs=[_row_spec(), pl.BlockSpec((_BLK, 1), lambda i: (i, 0)),
                  _full_spec((F, F))],
        out_specs=[_row_spec(), _row_spec()],
        out_shape=[jax.ShapeDtypeStruct((NP, F), jnp.float32),
                   jax.ShapeDtypeStruct((NP, F), jnp.float32)],
    )(h2, s, wrel3)


def _k45_body(p0_ref, p1_ref, r_ref, keep_ref,
              batch_ref, counts_ref, w1_ref, b1_ref, w2_ref, b2_ref,
              o_ref, gsum_ref):
    @pl.when(pl.program_id(0) == 0)
    def _init():
        gsum_ref[...] = jnp.zeros_like(gsum_ref)

    h3 = keep_ref[...] * jnp.maximum(
        p0_ref[0] + p1_ref[0] + r_ref[...], 0.0)
    oh = (batch_ref[...] == lax.broadcasted_iota(
        jnp.int32, (_BLK, N_GRAPHS), 1)).astype(jnp.float32)
    gsum_ref[...] += lax.dot_general(
        oh, h3, (((0,), (0,)), ((), ())),
        preferred_element_type=jnp.float32)

    @pl.when(pl.program_id(0) == _GRID - 1)
    def _head():
        mean = gsum_ref[...] / jnp.maximum(counts_ref[...], 1.0)
        z = jnp.maximum(jnp.dot(mean, w1_ref[...],
                                preferred_element_type=jnp.float32)
                        + b1_ref[...], 0.0)
        logits = jnp.dot(z, w2_ref[...],
                         preferred_element_type=jnp.float32) + b2_ref[...]
        mx = jnp.max(logits, axis=1, keepdims=True)
        lse = jnp.log(jnp.sum(jnp.exp(logits - mx), axis=1, keepdims=True))
        o_ref[...] = logits - mx - lse


def _final_conv_pool_head(part, r, keep, batch_col,
                          counts, w1, b1, w2, b2):
    return pl.pallas_call(
        _k45_body,
        grid=(_GRID,),
        in_specs=[_part_spec(0), _part_spec(1), _row_spec(),
                  pl.BlockSpec((_BLK, 1), lambda i: (i, 0)),
                  pl.BlockSpec((_BLK, 1), lambda i: (i, 0)),
                  _full_spec((N_GRAPHS, 1)),
                  _full_spec((F, 64)), _full_spec((1, 64)),
                  _full_spec((64, 10)), _full_spec((1, 10))],
        out_specs=_full_spec((N_GRAPHS, 10)),
        out_shape=jax.ShapeDtypeStruct((N_GRAPHS, 10), jnp.float32),
        scratch_shapes=[pltpu.VMEM((N_GRAPHS, F), jnp.float32)],
    )(part, part, r, keep, batch_col, counts, w1, b1, w2, b2)


def kernel(x, edge_index, batch,
           conv1_Wrel, conv1_brel, conv1_Wroot,
           conv2_Wrel, conv2_brel, conv2_Wroot,
           conv3_Wrel, conv3_brel, conv3_Wroot,
           pool_p, lin1_W, lin1_b, lin2_W, lin2_b):
    n = x.shape[0]
    xp = jnp.pad(x, ((0, NP - n), (0, 0)))
    src = edge_index[0].astype(jnp.int32)
    dst = edge_index[1].astype(jnp.int32)
    nfe = NFULL * CHUNK * NC * NS
    src2d = src[:nfe].reshape(NC * NS, NFULL, CHUNK)
    dst2d = dst[:nfe].reshape(NC * NS, NFULL, CHUNK)
    srct = src[nfe:]
    dstt = dst[nfe:]
    batch_pad = jnp.pad(batch.astype(jnp.int32), (0, NP - n),
                        constant_values=127)
    batch2 = batch_pad.reshape(NROWS, 128)
    batch_col = batch_pad.reshape(NP, 1)
    b1 = conv1_brel.reshape(1, F)
    b2 = conv2_brel.reshape(1, F)
    b3 = conv3_brel.reshape(1, F)
    pp_row = pool_p.reshape(1, F)
    pcol = pool_p.reshape(F, 1)

    y1 = _matmul(xp, conv1_Wrel)
    part1 = _sc_segment_sum(y1, src2d, dst2d, srct, dstt)
    r1 = _root(xp, conv1_Wroot, b1)
    h1, y2 = _combine_and_next(part1, r1, conv2_Wrel)

    part2 = _sc_segment_sum(y2, src2d, dst2d, srct, dstt)
    r2 = _root(h1, conv2_Wroot, b2)
    h2, q = _combine_and_score(part2, r2, pcol)

    s2, keep2, counts = _pool_select(q.reshape(NROWS, 128), batch2, pp_row)
    s_col = s2.reshape(NP, 1)
    keep_col = keep2.reshape(NP, 1)

    h2p, y3 = _scale_and_next(h2, s_col, conv3_Wrel)
    part3 = _sc_segment_sum(y3, src2d, dst2d, srct, dstt)
    r3 = _root(h2p, conv3_Wroot, b3)
    return _final_conv_pool_head(part3, r3,
                                 keep_col, batch_col, counts,
                                 lin1_W, lin1_b.reshape(1, 64),
                                 lin2_W, lin2_b.reshape(1, 10))


# speedup vs baseline: 1.2229x; 1.0205x over previous
import functools

import jax
import jax.numpy as jnp
from jax import lax
from jax.experimental import pallas as pl
from jax.experimental.pallas import tpu as pltpu
from jax.experimental.pallas import tpu_sc as plsc

N_GRAPHS = 16
NP = 10240
NROWS = NP // 128
F = 128
E = 320000
NC, NS = 2, 16
EDGES_PER_TILE = E // (NC * NS)
CHUNK = 128
NFULL = EDGES_PER_TILE // CHUNK
REM = EDGES_PER_TILE - NFULL * CHUNK
ROWS_PER_TILE = NP // NS
STAGE = 24
INT_MIN32 = -2147483648


def _sc_segment_sum(y, src2d, dst2d, srct, dstt):
    mesh = plsc.VectorSubcoreMesh(
        core_axis_name="c", subcore_axis_name="s", num_cores=NC,
        num_subcores=NS)

    @functools.partial(
        pl.kernel,
        out_type=jax.ShapeDtypeStruct((NC, NP, F), jnp.float32),
        mesh=mesh,
        scratch_types=[
            pltpu.VMEM((STAGE, CHUNK), jnp.int32),
            pltpu.VMEM((STAGE, CHUNK), jnp.int32),
            pltpu.VMEM((CHUNK, F), jnp.float32),
            pltpu.VMEM((CHUNK, F), jnp.float32),
            pltpu.VMEM((REM,), jnp.int32),
            pltpu.VMEM((REM,), jnp.int32),
            pltpu.VMEM((16, F), jnp.float32),
            pltpu.VMEM_SHARED((NP, F), jnp.float32),
            pltpu.SemaphoreType.DMA,
            pltpu.SemaphoreType.DMA,
            pltpu.SemaphoreType.DMA,
            pltpu.SemaphoreType.DMA,
            pltpu.SemaphoreType.DMA,
        ],
    )
    def k(y_hbm, src_hbm, dst_hbm, srct_hbm, dstt_hbm, out_hbm,
          sidx, didx, rows0, rows1, sidxr, didxr, zbuf, acc,
          sg0, sg1, ss0, ss1, semi):
        c = lax.axis_index("c")
        s = lax.axis_index("s")
        w = c * NS + s

        i0 = pltpu.make_async_copy(src_hbm.at[w, pl.ds(0, STAGE)], sidx, semi)
        i1 = pltpu.make_async_copy(dst_hbm.at[w, pl.ds(0, STAGE)], didx, semi)
        i2 = pltpu.make_async_copy(srct_hbm.at[pl.ds(w * REM, REM)],
                                   sidxr, semi)
        i3 = pltpu.make_async_copy(dstt_hbm.at[pl.ds(w * REM, REM)],
                                   didxr, semi)
        i0.start(); i1.start(); i2.start(); i3.start()

        for i in range(16):
            for j in range(F // 16):
                zbuf[i, pl.ds(j * 16, 16)] = jnp.zeros((16,), jnp.float32)
        row0 = s * ROWS_PER_TILE

        def zero_body(kk, _):
            pltpu.sync_copy(zbuf, acc.at[pl.ds(row0 + kk * 16, 16)])
            return _
        lax.fori_loop(0, ROWS_PER_TILE // 16, zero_body, 0)
        i0.wait(); i1.wait(); i2.wait(); i3.wait()
        plsc.subcore_barrier()

        def gather(kk, buf, sg):
            return pltpu.make_async_copy(y_hbm.at[sidx.at[kk]], buf, sg)

        def run_stage(npairs):
            gather(0, rows0, sg0).start()

            def pair_body(kk, _):
                e = 2 * kk
                gather(e, rows0, sg0).wait()
                gather(e + 1, rows1, sg1).start()
                pltpu.sync_copy(rows0, acc.at[didx.at[e]], add=True)
                gather(e + 1, rows1, sg1).wait()

                @pl.when(kk < npairs - 1)
                def _next():
                    gather(e + 2, rows0, sg0).start()

                pltpu.sync_copy(rows1, acc.at[didx.at[e + 1]], add=True)
                return _
            lax.fori_loop(0, npairs, pair_body, 0)

        for t in range(NFULL // STAGE):
            if t > 0:
                pltpu.sync_copy(src_hbm.at[w, pl.ds(t * STAGE, STAGE)], sidx)
                pltpu.sync_copy(dst_hbm.at[w, pl.ds(t * STAGE, STAGE)], didx)
            run_stage(STAGE // 2)
        tcount = NFULL - (NFULL // STAGE) * STAGE
        pltpu.sync_copy(
            src_hbm.at[w, pl.ds((NFULL // STAGE) * STAGE, tcount)],
            sidx.at[pl.ds(0, tcount)])
        pltpu.sync_copy(
            dst_hbm.at[w, pl.ds((NFULL // STAGE) * STAGE, tcount)],
            didx.at[pl.ds(0, tcount)])
        run_stage(tcount // 2)

        pltpu.async_copy(y_hbm.at[sidxr], rows1.at[pl.ds(0, REM)], sg1).wait()
        pltpu.sync_copy(rows1.at[pl.ds(0, REM)], acc.at[didxr], add=True)

        plsc.subcore_barrier()
        pltpu.sync_copy(acc.at[pl.ds(row0, ROWS_PER_TILE)],
                        out_hbm.at[c].at[pl.ds(row0, ROWS_PER_TILE)])

    return k(y, src2d, dst2d, srct, dstt)


_BLK = 5120
_GRID = NP // _BLK


def _row_spec(blk=_BLK, width=F):
    return pl.BlockSpec((blk, width), lambda i: (i, 0))


def _part_spec(core):
    return pl.BlockSpec((1, _BLK, F), lambda i, c=core: (c, i, 0))


def _full_spec(shape):
    return pl.BlockSpec(shape, lambda i: tuple(0 for _ in shape))


def _k0_body(x_ref, w_ref, o_ref):
    o_ref[...] = jnp.dot(x_ref[...], w_ref[...],
                         preferred_element_type=jnp.float32)


def _matmul(x, w):
    return pl.pallas_call(
        _k0_body,
        grid=(_GRID,),
        in_specs=[_row_spec(), _full_spec((F, F))],
        out_specs=_row_spec(),
        out_shape=jax.ShapeDtypeStruct((NP, F), jnp.float32),
    )(x, w)


def _kr_body(h_ref, wroot_ref, b_ref, r_ref):
    r_ref[...] = jnp.dot(h_ref[...], wroot_ref[...],
                         preferred_element_type=jnp.float32) + b_ref[...]


def _root(h, wroot, b):
    return pl.pallas_call(
        _kr_body,
        grid=(_GRID,),
        in_specs=[_row_spec(), _full_spec((F, F)), _full_spec((1, F))],
        out_specs=_row_spec(),
        out_shape=jax.ShapeDtypeStruct((NP, F), jnp.float32),
    )(h, wroot, b)


def _k1_body(p0_ref, p1_ref, r_ref, wrel2_ref, h1_ref, y2_ref):
    h1 = jnp.maximum(p0_ref[0] + p1_ref[0] + r_ref[...], 0.0)
    h1_ref[...] = h1
    y2_ref[...] = jnp.dot(h1, wrel2_ref[...],
                          preferred_element_type=jnp.float32)


def _combine_and_next(part, r, wrel2):
    return pl.pallas_call(
        _k1_body,
        grid=(_GRID,),
        in_specs=[_part_spec(0), _part_spec(1), _row_spec(),
                  _full_spec((F, F))],
        out_specs=[_row_spec(), _row_spec()],
        out_shape=[jax.ShapeDtypeStruct((NP, F), jnp.float32),
                   jax.ShapeDtypeStruct((NP, F), jnp.float32)],
    )(part, part, r, wrel2)


def _k2_body(p0_ref, p1_ref, r_ref, pcol_ref, h2_ref, q_ref):
    h2 = jnp.maximum(p0_ref[0] + p1_ref[0] + r_ref[...], 0.0)
    h2_ref[...] = h2
    q_ref[...] = jnp.dot(h2, pcol_ref[...],
                         preferred_element_type=jnp.float32)


def _combine_and_score(part, r, pcol):
    return pl.pallas_call(
        _k2_body,
        grid=(_GRID,),
        in_specs=[_part_spec(0), _part_spec(1), _row_spec(),
                  _full_spec((F, 1))],
        out_specs=[_row_spec(), pl.BlockSpec((_BLK, 1), lambda i: (i, 0))],
        out_shape=[jax.ShapeDtypeStruct((NP, F), jnp.float32),
                   jax.ShapeDtypeStruct((NP, 1), jnp.float32)],
    )(part, part, r, pcol)


def _pool_body(q_ref, batch_ref, pp_ref, s_ref, keep_ref, counts_ref):
    q = q_ref[...]
    batch = batch_ref[...]
    bits = lax.bitcast_convert_type(q, jnp.int32)
    key = bits ^ (lax.shift_right_arithmetic(bits, 31) & jnp.int32(0x7FFFFFFF))

    mg = [batch == g for g in range(N_GRAPHS)]
    mgf = [m.astype(jnp.float32) for m in mg]
    ones_col = jnp.ones((128, 1), jnp.float32)

    def select_chain(scalars):
        t = jnp.broadcast_to(scalars[N_GRAPHS - 1], q.shape)
        for g in range(N_GRAPHS - 1):
            t = jnp.where(mg[g], scalars[g], t)
        return t

    def gcounts(predf, masksf):
        rows = [jnp.sum(predf * masksf[g], axis=0, keepdims=True)
                for g in range(N_GRAPHS)]
        return lax.dot_general(jnp.concatenate(rows, axis=0), ones_col,
                               (((1,), (0,)), ((), ())),
                               preferred_element_type=jnp.float32)

    onesf = jnp.ones(q.shape, jnp.float32)
    sizes = gcounts(onesf, mgf)
    kper = jnp.ceil(jnp.float32(0.8) * sizes)
    kper_i = kper.astype(jnp.int32)

    def t_body(i, tu):
        b = 31 - i
        bit = lax.shift_left(jnp.int32(1), b)
        cand_u = tu | bit
        cand_k = cand_u ^ jnp.int32(INT_MIN32)
        tn = select_chain([cand_k[g, 0] for g in range(N_GRAPHS)])
        predf = (key >= tn).astype(jnp.float32)
        cnt = gcounts(predf, mgf)
        return jnp.where(cnt >= kper, cand_u, tu)

    tu = lax.fori_loop(0, 32, t_body, jnp.zeros((N_GRAPHS, 1), jnp.int32))
    tk = tu ^ jnp.int32(INT_MIN32)

    tkn = select_chain([tk[g, 0] for g in range(N_GRAPHS)])
    tief = (key == tkn).astype(jnp.float32)
    wgf = [tief * mgf[g] for g in range(N_GRAPHS)]
    cgt = gcounts((key > tkn).astype(jnp.float32), mgf)
    m = kper - cgt

    idx = (lax.broadcasted_iota(jnp.int32, q.shape, 0) * 128
           + lax.broadcasted_iota(jnp.int32, q.shape, 1))

    def j_body(i, jv):
        b = 13 - i
        bit = lax.shift_left(jnp.int32(1), b)
        cand = jv | bit
        jn = select_chain([cand[g, 0] for g in range(N_GRAPHS)])
        cnt = gcounts((idx < jn).astype(jnp.float32), wgf)
        return jnp.where(cnt < m, cand, jv)

    jv = lax.fori_loop(0, 14, j_body, jnp.zeros((N_GRAPHS, 1), jnp.int32))

    jn = select_chain([jv[g, 0] for g in range(N_GRAPHS)])
    keep = (key > tkn) | ((key == tkn) & (idx <= jn))
    keep = keep & (batch < N_GRAPHS)
    keep_f = keep.astype(jnp.float32)

    pp = pp_ref[...]
    inv_norm = lax.rsqrt(jnp.sum(pp * pp))
    s_ref[...] = jnp.tanh(q * inv_norm) * keep_f
    keep_ref[...] = keep_f
    counts_ref[...] = gcounts(keep_f, mgf)


def _pool_select(q2, batch2, pp):
    return pl.pallas_call(
        _pool_body,
        grid=(1,),
        in_specs=[_full_spec((NROWS, 128)), _full_spec((NROWS, 128)),
                  _full_spec((1, F))],
        out_specs=[_full_spec((NROWS, 128)), _full_spec((NROWS, 128)),
                   _full_spec((N_GRAPHS, 1))],
        out_shape=[jax.ShapeDtypeStruct((NROWS, 128), jnp.float32),
                   jax.ShapeDtypeStruct((NROWS, 128), jnp.float32),
                   jax.ShapeDtypeStruct((N_GRAPHS, 1), jnp.float32)],
    )(q2, batch2, pp)


def _k3b_body(h2_ref, s_ref, wrel3_ref, h2p_ref, y3_ref):
    h2p = h2_ref[...] * s_ref[...]
    h2p_ref[...] = h2p
    y3_ref[...] = jnp.dot(h2p, wrel3_ref[...],
                          preferred_element_type=jnp.float32)


def _scale_and_next(h2, s, wrel3):
    return pl.pallas_call(
        _k3b_body,
        grid=(_GRID,),
        in_specs=[_row_spec(), pl.BlockSpec((_BLK, 1), lambda i: (i, 0)),
                  _full_spec((F, F))],
        out_specs=[_row_spec(), _row_spec()],
        out_shape=[jax.ShapeDtypeStruct((NP, F), jnp.float32),
                   jax.ShapeDtypeStruct((NP, F), jnp.float32)],
    )(h2, s, wrel3)


def _k45_body(p0_ref, p1_ref, r_ref, keep_ref,
              batch_ref, counts_ref, w1_ref, b1_ref, w2_ref, b2_ref,
              o_ref, gsum_ref):
    @pl.when(pl.program_id(0) == 0)
    def _init():
        gsum_ref[...] = jnp.zeros_like(gsum_ref)

    h3 = keep_ref[...] * jnp.maximum(
        p0_ref[0] + p1_ref[0] + r_ref[...], 0.0)
    oh = (batch_ref[...] == lax.broadcasted_iota(
        jnp.int32, (_BLK, N_GRAPHS), 1)).astype(jnp.float32)
    gsum_ref[...] += lax.dot_general(
        oh, h3, (((0,), (0,)), ((), ())),
        preferred_element_type=jnp.float32)

    @pl.when(pl.program_id(0) == _GRID - 1)
    def _head():
        mean = gsum_ref[...] / jnp.maximum(counts_ref[...], 1.0)
        z = jnp.maximum(jnp.dot(mean, w1_ref[...],
                                preferred_element_type=jnp.float32)
                        + b1_ref[...], 0.0)
        logits = jnp.dot(z, w2_ref[...],
                         preferred_element_type=jnp.float32) + b2_ref[...]
        mx = jnp.max(logits, axis=1, keepdims=True)
        lse = jnp.log(jnp.sum(jnp.exp(logits - mx), axis=1, keepdims=True))
        o_ref[...] = logits - mx - lse


def _final_conv_pool_head(part, r, keep, batch_col,
                          counts, w1, b1, w2, b2):
    return pl.pallas_call(
        _k45_body,
        grid=(_GRID,),
        in_specs=[_part_spec(0), _part_spec(1), _row_spec(),
                  pl.BlockSpec((_BLK, 1), lambda i: (i, 0)),
                  pl.BlockSpec((_BLK, 1), lambda i: (i, 0)),
                  _full_spec((N_GRAPHS, 1)),
                  _full_spec((F, 64)), _full_spec((1, 64)),
                  _full_spec((64, 10)), _full_spec((1, 10))],
        out_specs=_full_spec((N_GRAPHS, 10)),
        out_shape=jax.ShapeDtypeStruct((N_GRAPHS, 10), jnp.float32),
        scratch_shapes=[pltpu.VMEM((N_GRAPHS, F), jnp.float32)],
    )(part, part, r, keep, batch_col, counts, w1, b1, w2, b2)


def kernel(x, edge_index, batch,
           conv1_Wrel, conv1_brel, conv1_Wroot,
           conv2_Wrel, conv2_brel, conv2_Wroot,
           conv3_Wrel, conv3_brel, conv3_Wroot,
           pool_p, lin1_W, lin1_b, lin2_W, lin2_b):
    n = x.shape[0]
    xp = jnp.pad(x, ((0, NP - n), (0, 0)))
    src = edge_index[0].astype(jnp.int32)
    dst = edge_index[1].astype(jnp.int32)
    nfe = NFULL * CHUNK * NC * NS
    src2d = src[:nfe].reshape(NC * NS, NFULL, CHUNK)
    dst2d = dst[:nfe].reshape(NC * NS, NFULL, CHUNK)
    srct = src[nfe:]
    dstt = dst[nfe:]
    batch_pad = jnp.pad(batch.astype(jnp.int32), (0, NP - n),
                        constant_values=127)
    batch2 = batch_pad.reshape(NROWS, 128)
    batch_col = batch_pad.reshape(NP, 1)
    b1 = conv1_brel.reshape(1, F)
    b2 = conv2_brel.reshape(1, F)
    b3 = conv3_brel.reshape(1, F)
    pp_row = pool_p.reshape(1, F)
    pcol = pool_p.reshape(F, 1)

    y1 = _matmul(xp, conv1_Wrel)
    part1 = _sc_segment_sum(y1, src2d, dst2d, srct, dstt)
    r1 = _root(xp, conv1_Wroot, b1)
    h1, y2 = _combine_and_next(part1, r1, conv2_Wrel)

    part2 = _sc_segment_sum(y2, src2d, dst2d, srct, dstt)
    r2 = _root(h1, conv2_Wroot, b2)
    h2, q = _combine_and_score(part2, r2, pcol)

    s2, keep2, counts = _pool_select(q.reshape(NROWS, 128), batch2, pp_row)
    s_col = s2.reshape(NP, 1)
    keep_col = keep2.reshape(NP, 1)

    h2p, y3 = _scale_and_next(h2, s_col, conv3_Wrel)
    part3 = _sc_segment_sum(y3, src2d, dst2d, srct, dstt)
    r3 = _root(h2p, conv3_Wroot, b3)
    return _final_conv_pool_head(part3, r3,
                                 keep_col, batch_col, counts,
                                 lin1_W, lin1_b.reshape(1, 64),
                                 lin2_W, lin2_b.reshape(1, 10))
